# Initial kernel scaffold; baseline (speedup 1.0000x reference)
import jax
import jax.numpy as jnp
from jax.experimental import pallas as pl


def kernel(inp, W):
    raise NotImplementedError("write your pallas kernel here")


# speedup vs baseline: 4.2877x; 4.2877x over previous
import functools
import math

import jax
import jax.numpy as jnp
from jax import lax
from jax.experimental import pallas as pl
from jax.experimental.pallas import tpu as pltpu
from jax.experimental.pallas import tpu_sc as plsc

_VOCAB = 100000
_D = 128
_BATCH = 1024
_SEQ = 200
_SCALE = math.sqrt(float(_D))

_NW = 32
_B_TOTAL = _BATCH * _SEQ
_ROWS_PER_W = _B_TOTAL // _NW
_CHUNK = 128
_CHUNKS_PER_W = _ROWS_PER_W // _CHUNK

_W_BLOCK = 2000


def _scale_mask_body(w_ref, inp_ref, ws_ref, mask_ref):
    i = pl.program_id(0)
    ws_ref[...] = w_ref[...] * jnp.float32(_SCALE)

    @pl.when(i == 0)
    def _():
        ws_ref[0:1, :] = jnp.zeros((1, _D), jnp.float32)
        mask_ref[...] = inp_ref[...] == 0


def _scale_and_mask(W, inp):
    n_blocks = _VOCAB // _W_BLOCK
    return pl.pallas_call(
        _scale_mask_body,
        grid=(n_blocks,),
        in_specs=[
            pl.BlockSpec((_W_BLOCK, _D), lambda i: (i, 0)),
            pl.BlockSpec((_BATCH, _SEQ), lambda i: (0, 0)),
        ],
        out_specs=[
            pl.BlockSpec((_W_BLOCK, _D), lambda i: (i, 0)),
            pl.BlockSpec((_BATCH, _SEQ), lambda i: (0, 0)),
        ],
        out_shape=[
            jax.ShapeDtypeStruct((_VOCAB, _D), jnp.float32),
            jax.ShapeDtypeStruct((_BATCH, _SEQ), jnp.bool_),
        ],
    )(W, inp)


def _sc_gather_body(table_hbm, idx_hbm, out_hbm, idx_v, rows_v, sem):
    wid = lax.axis_index("s") * 2 + lax.axis_index("c")
    base = wid * _ROWS_PER_W
    pltpu.sync_copy(idx_hbm.at[wid], idx_v)

    def body(j, _):
        pltpu.async_copy(table_hbm.at[idx_v.at[j]], rows_v, sem).wait()
        pltpu.sync_copy(rows_v, out_hbm.at[pl.ds(base + j * _CHUNK, _CHUNK)])
        return 0

    lax.fori_loop(0, _CHUNKS_PER_W, body, 0)


_sc_gather = functools.partial(
    pl.kernel,
    out_type=jax.ShapeDtypeStruct((_B_TOTAL, _D), jnp.float32),
    mesh=plsc.VectorSubcoreMesh(core_axis_name="c", subcore_axis_name="s"),
    scratch_types=[
        pltpu.VMEM((_CHUNKS_PER_W, _CHUNK), jnp.int32),
        pltpu.VMEM((_CHUNK, _D), jnp.float32),
        pltpu.SemaphoreType.DMA,
    ],
)(_sc_gather_body)


def kernel(inp, W):
    Ws, mask = _scale_and_mask(W, inp)
    idx2d = jnp.transpose(inp).reshape(_NW, _CHUNKS_PER_W, _CHUNK)
    flat = _sc_gather(Ws, idx2d)
    return flat.reshape(_SEQ, _BATCH, _D), mask


# speedup vs baseline: 5.3354x; 1.2443x over previous
import functools
import math

import jax
import jax.numpy as jnp
from jax import lax
from jax.experimental import pallas as pl
from jax.experimental.pallas import tpu as pltpu
from jax.experimental.pallas import tpu_sc as plsc

_VOCAB = 100000
_D = 128
_BATCH = 1024
_SEQ = 200
_SCALE = math.sqrt(float(_D))

_NW = 32
_B_TOTAL = _BATCH * _SEQ
_ROWS_PER_W = _B_TOTAL // _NW
_CHUNK = 128
_CHUNKS_PER_W = _ROWS_PER_W // _CHUNK

_W_BLOCK = 2000


def _scale_mask_body(w_ref, inp_ref, ws_ref, mask_ref):
    i = pl.program_id(0)
    ws_ref[...] = w_ref[...] * jnp.float32(_SCALE)

    @pl.when(i == 0)
    def _():
        ws_ref[0:1, :] = jnp.zeros((1, _D), jnp.float32)
        mask_ref[...] = inp_ref[...] == 0


def _scale_and_mask(W, inp):
    n_blocks = _VOCAB // _W_BLOCK
    return pl.pallas_call(
        _scale_mask_body,
        grid=(n_blocks,),
        in_specs=[
            pl.BlockSpec((_W_BLOCK, _D), lambda i: (i, 0)),
            pl.BlockSpec((_BATCH, _SEQ), lambda i: (0, 0)),
        ],
        out_specs=[
            pl.BlockSpec((_W_BLOCK, _D), lambda i: (i, 0)),
            pl.BlockSpec((_BATCH, _SEQ), lambda i: (0, 0)),
        ],
        out_shape=[
            jax.ShapeDtypeStruct((_VOCAB, _D), jnp.float32),
            jax.ShapeDtypeStruct((_BATCH, _SEQ), jnp.bool_),
        ],
    )(W, inp)


_NBUF = 5
_ROUNDS = _CHUNKS_PER_W // _NBUF


def _sc_gather_body(table_hbm, idx_hbm, out_hbm, idx_v, rows_v, *sems):
    gsem = sems[:_NBUF]
    osem = sems[_NBUF:]
    wid = lax.axis_index("s") * 2 + lax.axis_index("c")
    base = wid * _ROWS_PER_W
    pltpu.sync_copy(idx_hbm.at[wid], idx_v)

    def g_desc(i, j):
        return pltpu.make_async_copy(
            table_hbm.at[idx_v.at[j]], rows_v.at[i], gsem[i])

    def o_desc(i, j):
        return pltpu.make_async_copy(
            rows_v.at[i], out_hbm.at[pl.ds(base + j * _CHUNK, _CHUNK)],
            osem[i])

    for i in range(_NBUF):
        g_desc(i, i).start()

    def body(k, _):
        for i in range(_NBUF):
            j = k * _NBUF + i
            g_desc(i, j).wait()
            o_desc(i, j).start()

            def refill(i=i, j=j):
                o_desc(i, j).wait()
                g_desc(i, j + _NBUF).start()

            pl.when(k < _ROUNDS - 1)(refill)
        return 0

    lax.fori_loop(0, _ROUNDS, body, 0)
    for i in range(_NBUF):
        o_desc(i, (_ROUNDS - 1) * _NBUF + i).wait()


_sc_gather = functools.partial(
    pl.kernel,
    out_type=jax.ShapeDtypeStruct((_B_TOTAL, _D), jnp.float32),
    mesh=plsc.VectorSubcoreMesh(core_axis_name="c", subcore_axis_name="s"),
    scratch_types=[
        pltpu.VMEM((_CHUNKS_PER_W, _CHUNK), jnp.int32),
        pltpu.VMEM((_NBUF, _CHUNK, _D), jnp.float32),
    ] + [pltpu.SemaphoreType.DMA] * (2 * _NBUF),
)(_sc_gather_body)


def kernel(inp, W):
    Ws, mask = _scale_and_mask(W, inp)
    idx2d = jnp.transpose(inp).reshape(_NW, _CHUNKS_PER_W, _CHUNK)
    flat = _sc_gather(Ws, idx2d)
    return flat.reshape(_SEQ, _BATCH, _D), mask
